# Initial kernel scaffold; baseline (speedup 1.0000x reference)
import jax
import jax.numpy as jnp
from jax.experimental import pallas as pl


def kernel(feats, edge_index, W, b, W_res, b_res, gamma, beta):
    raise NotImplementedError("write your pallas kernel here")


# speedup vs baseline: 4.1985x; 4.1985x over previous
import functools

import jax
import jax.numpy as jnp
from jax import lax
from jax.experimental import pallas as pl
from jax.experimental.pallas import tpu as pltpu
from jax.experimental.pallas import tpu_sc as plsc

N = 10000
E = 320000
D = 128
EPS = 1e-5

NC = 2
NS = 16
NW = NC * NS
C = 128

CHUNKS_PER_W = -(-E // (NW * C))
EPW = CHUNKS_PER_W * C
EP = EPW * NW
NP = 10112
INIT_ROWS = NP // NS
OUT_ROWS = 632
OUT_TAIL = N - 15 * OUT_ROWS


def _sc_segment_sum(src_p, dst_p, feats, zeros):
    mesh = plsc.VectorSubcoreMesh(core_axis_name="c", subcore_axis_name="s")

    @functools.partial(
        pl.kernel,
        out_type=jax.ShapeDtypeStruct((NC, N, D), jnp.float32),
        mesh=mesh,
        scratch_types=[
            pltpu.VMEM((C,), jnp.int32),
            pltpu.VMEM((C,), jnp.int32),
            pltpu.VMEM((C, D), jnp.float32),
            pltpu.VMEM_SHARED((NP, D), jnp.float32),
            pltpu.SemaphoreType.DMA,
        ],
    )
    def seg_sum(src_hbm, dst_hbm, feats_hbm, zeros_hbm, out_hbm,
                src_v, dst_v, rows_v, acc_sh, sem):
        cid = lax.axis_index("c")
        sid = lax.axis_index("s")
        pltpu.sync_copy(zeros_hbm.at[pl.ds(sid * INIT_ROWS, INIT_ROWS)],
                        acc_sh.at[pl.ds(sid * INIT_ROWS, INIT_ROWS)])
        plsc.subcore_barrier()

        wid = sid * NC + cid
        base = wid * EPW

        @pl.loop(0, CHUNKS_PER_W)
        def _(j):
            off = base + j * C
            pltpu.sync_copy(src_hbm.at[pl.ds(off, C)], src_v)
            pltpu.sync_copy(dst_hbm.at[pl.ds(off, C)], dst_v)
            pltpu.async_copy(feats_hbm.at[src_v], rows_v, sem).wait()
            pltpu.sync_copy(rows_v, acc_sh.at[dst_v], add=True)

        plsc.subcore_barrier()

        @pl.when(sid < NS - 1)
        def _():
            pltpu.sync_copy(acc_sh.at[pl.ds(sid * OUT_ROWS, OUT_ROWS)],
                            out_hbm.at[cid, pl.ds(sid * OUT_ROWS, OUT_ROWS)])

        @pl.when(sid == NS - 1)
        def _():
            pltpu.sync_copy(acc_sh.at[pl.ds((NS - 1) * OUT_ROWS, OUT_TAIL)],
                            out_hbm.at[cid, pl.ds((NS - 1) * OUT_ROWS, OUT_TAIL)])

    return seg_sum(src_p, dst_p, feats, zeros)


R = 1000
NBLK = N // R


def _tc_fused_body(p0_ref, p1_ref, f_ref, w_ref, b_ref, wr_ref, br_ref,
                   h_ref, stats_ref, acc_ref):
    i = pl.program_id(0)
    agg = p0_ref[...] + p1_ref[...]
    h = jnp.dot(agg, w_ref[...], preferred_element_type=jnp.float32)
    h = jnp.maximum(h + b_ref[...], 0.0)
    r = jnp.dot(f_ref[...], wr_ref[...], preferred_element_type=jnp.float32)
    r = jnp.maximum(r + br_ref[...], 0.0)
    h = h + r
    h_ref[...] = h

    @pl.when(i == 0)
    def _():
        acc_ref[...] = jnp.zeros_like(acc_ref)

    acc_ref[0:1, :] += jnp.sum(h, axis=0, keepdims=True)
    acc_ref[1:2, :] += jnp.sum(h * h, axis=0, keepdims=True)

    @pl.when(i == NBLK - 1)
    def _():
        stats_ref[...] = acc_ref[...]


def _tc_norm_body(h_ref, stats_ref, g_ref, bt_ref, o_ref):
    mean = stats_ref[0:1, :] * (1.0 / N)
    var = stats_ref[1:2, :] * (1.0 / N) - mean * mean
    inv = lax.rsqrt(var + EPS)
    o_ref[...] = (h_ref[...] - mean) * (inv * g_ref[...]) + bt_ref[...]


def kernel(feats, edge_index, W, b, W_res, b_res, gamma, beta):
    src = edge_index[0].astype(jnp.int32)
    dst = edge_index[1].astype(jnp.int32)
    pad = EP - E
    src_p = jnp.concatenate([src, jnp.zeros((pad,), jnp.int32)])
    dst_p = jnp.concatenate([dst, jnp.full((pad,), N, jnp.int32)])
    zeros = jnp.zeros((NP, D), jnp.float32)

    parts = _sc_segment_sum(src_p, dst_p, feats, zeros)
    p0, p1 = parts[0], parts[1]

    blk = lambda i: (i, 0)
    full = lambda i: (0, 0)
    h_pre, stats = pl.pallas_call(
        _tc_fused_body,
        grid=(NBLK,),
        in_specs=[
            pl.BlockSpec((R, D), blk),
            pl.BlockSpec((R, D), blk),
            pl.BlockSpec((R, D), blk),
            pl.BlockSpec((D, D), full),
            pl.BlockSpec((1, D), full),
            pl.BlockSpec((D, D), full),
            pl.BlockSpec((1, D), full),
        ],
        out_specs=[
            pl.BlockSpec((R, D), blk),
            pl.BlockSpec((2, D), full),
        ],
        out_shape=[
            jax.ShapeDtypeStruct((N, D), jnp.float32),
            jax.ShapeDtypeStruct((2, D), jnp.float32),
        ],
        scratch_shapes=[pltpu.VMEM((2, D), jnp.float32)],
    )(p0, p1, feats, W, b.reshape(1, D), W_res, b_res.reshape(1, D))

    out = pl.pallas_call(
        _tc_norm_body,
        grid=(NBLK,),
        in_specs=[
            pl.BlockSpec((R, D), blk),
            pl.BlockSpec((2, D), full),
            pl.BlockSpec((1, D), full),
            pl.BlockSpec((1, D), full),
        ],
        out_specs=pl.BlockSpec((R, D), blk),
        out_shape=jax.ShapeDtypeStruct((N, D), jnp.float32),
    )(h_pre, stats, gamma.reshape(1, D), beta.reshape(1, D))
    return out
